# baseline (speedup 1.0000x reference)
import functools

import jax
import jax.numpy as jnp
from jax import lax
from jax.experimental import pallas as pl
from jax.experimental.pallas import tpu as pltpu
from jax.experimental.pallas import tpu_sc as plsc

_VOCAB = 100000
_EMBED_DIM = 128
_N_ROWS = 64
_STRIDE = 1562
_LANES = 16


def _sc_gather(x):
    mesh = plsc.VectorSubcoreMesh(
        core_axis_name="c", subcore_axis_name="s", num_cores=1, num_subcores=1
    )

    @functools.partial(
        pl.kernel,
        mesh=mesh,
        out_type=jax.ShapeDtypeStruct((_N_ROWS, _EMBED_DIM), jnp.float32),
        scratch_types=[
            pltpu.VMEM((_N_ROWS,), jnp.int32),
            pltpu.VMEM((_N_ROWS, _EMBED_DIM), jnp.float32),
            pltpu.SemaphoreType.DMA,
        ],
    )
    def k(x_hbm, out_hbm, idx_v, rows_v, sem):
        lanes = lax.iota(jnp.int32, _LANES)
        for q in range(_N_ROWS // _LANES):
            idx_v[pl.ds(q * _LANES, _LANES)] = (q * _LANES + lanes) * _STRIDE
        pltpu.async_copy(x_hbm.at[idx_v], rows_v, sem).wait()
        pltpu.sync_copy(rows_v, out_hbm)

    return k(x)


def kernel(x):
    return _sc_gather(x)


# speedup vs baseline: 1.0991x; 1.0991x over previous
import functools

import jax
import jax.numpy as jnp
from jax.experimental import pallas as pl
from jax.experimental.pallas import tpu as pltpu
from jax.experimental.pallas import tpu_sc as plsc

_VOCAB = 100000
_EMBED_DIM = 128
_N_ROWS = 64
_STRIDE = 1562


def _sc_gather(x):
    mesh = plsc.ScalarSubcoreMesh(axis_name="c", num_cores=1)

    @functools.partial(
        pl.kernel,
        mesh=mesh,
        out_type=jax.ShapeDtypeStruct((_N_ROWS, _EMBED_DIM), jnp.float32),
        scratch_types=[
            pltpu.VMEM_SHARED((8, 8, _EMBED_DIM), jnp.float32),
            pltpu.SemaphoreType.DMA,
        ],
    )
    def k(x_hbm, out_hbm, sp, sem):
        x3 = x_hbm.at[pl.ds(0, _N_ROWS * _STRIDE)].reshape(
            8, 8 * _STRIDE, _EMBED_DIM
        )
        copies = [
            pltpu.async_copy(
                x3.at[:, pl.ds(r * _STRIDE, 1), :],
                sp.at[:, pl.ds(r, 1), :],
                sem,
            )
            for r in range(8)
        ]
        for c in copies:
            c.wait()
        pltpu.sync_copy(sp.reshape(_N_ROWS, _EMBED_DIM), out_hbm)

    return k(x)


def kernel(x):
    return _sc_gather(x)
